# baseline (speedup 1.0000x reference)
import functools

import jax
import jax.numpy as jnp
import numpy as np
from jax import lax
from jax.experimental import pallas as pl
from jax.experimental.pallas import tpu as pltpu
from jax.experimental.pallas import tpu_sc as plsc

_FIELD_DIMS = [100000] * 26
_OFFSETS = np.concatenate(([0], np.cumsum(_FIELD_DIMS)[:-1])).astype(np.int32)

_B = 4096
_F = 26
_D = 16
_NW = 32
_SPW = _B // _NW
_IDX_PER_W = _SPW * _F
_CHUNK = 128
_NCH = _IDX_PER_W // _CHUNK
_EMB_ROWS = 2600000
_PACK = 128 // _D


def _extract(j, buf, p_v, sc0_v, er_v, ec0_v, out_v):
    del p_v
    for grp in range(_CHUNK // 16):
        base = grp * 16
        srow = jnp.arange(16, dtype=jnp.int32) + base
        sc0 = sc0_v[j, pl.ds(base, 16)]
        er = er_v[j, pl.ds(base, 16)]
        ec0 = ec0_v[j, pl.ds(base, 16)]
        for d in range(_D):
            vals = plsc.load_gather(buf, [srow, sc0 + d])
            plsc.store_scatter(out_v, [er, ec0 + d], vals)


def _sc_gather(embr_hbm, lin_hbm, p_hbm, sc0_hbm, er_hbm, ec0_hbm, g_hbm,
               e_out, lv_out,
               p_v, sc0_v, er_v, ec0_v, g_v, buf0, buf1, out_v, lin_v,
               sem_g, sem_l):
    wid = lax.axis_index("s") * 2 + lax.axis_index("c")
    pltpu.sync_copy(p_hbm.at[wid], p_v)
    pltpu.sync_copy(sc0_hbm.at[wid], sc0_v)
    pltpu.sync_copy(er_hbm.at[wid], er_v)
    pltpu.sync_copy(ec0_hbm.at[wid], ec0_v)
    pltpu.sync_copy(g_hbm.at[wid], g_v)

    for j in range(_NCH):
        pltpu.make_async_copy(lin_hbm.at[g_v.at[j]], lin_v.at[j], sem_l).start()

    pltpu.make_async_copy(embr_hbm.at[p_v.at[0]], buf0, sem_g).start()

    def body(k, _):
        j = 2 * k
        pltpu.make_async_copy(embr_hbm.at[p_v.at[j + 1]], buf1, sem_g).start()
        pltpu.make_async_copy(embr_hbm.at[p_v.at[0]], buf0, sem_g).wait()
        _extract(j, buf0, p_v, sc0_v, er_v, ec0_v, out_v)

        @pl.when(k < _NCH // 2 - 1)
        def _():
            pltpu.make_async_copy(embr_hbm.at[p_v.at[j + 2]], buf0, sem_g).start()

        pltpu.make_async_copy(embr_hbm.at[p_v.at[0]], buf1, sem_g).wait()
        _extract(j + 1, buf1, p_v, sc0_v, er_v, ec0_v, out_v)
        return 0

    lax.fori_loop(0, _NCH // 2, body, 0)

    pltpu.make_async_copy(lv_out.at[wid], lin_v, sem_l).wait()
    pltpu.sync_copy(out_v, e_out.at[pl.ds(wid * _SPW, _SPW)])
    pltpu.sync_copy(lin_v, lv_out.at[wid])


_gather_call = functools.partial(
    pl.kernel,
    out_type=[
        jax.ShapeDtypeStruct((_B, _F * _D), jnp.float32),
        jax.ShapeDtypeStruct((_NW, _NCH, _CHUNK), jnp.float32),
    ],
    mesh=plsc.VectorSubcoreMesh(
        core_axis_name="c", subcore_axis_name="s", num_cores=2, num_subcores=16
    ),
    scratch_types=[
        pltpu.VMEM((_NCH, _CHUNK), jnp.int32),
        pltpu.VMEM((_NCH, _CHUNK), jnp.int32),
        pltpu.VMEM((_NCH, _CHUNK), jnp.int32),
        pltpu.VMEM((_NCH, _CHUNK), jnp.int32),
        pltpu.VMEM((_NCH, _CHUNK), jnp.int32),
        pltpu.VMEM((_CHUNK, 128), jnp.float32),
        pltpu.VMEM((_CHUNK, 128), jnp.float32),
        pltpu.VMEM((_SPW, _F * _D), jnp.float32),
        pltpu.VMEM((_NCH, _CHUNK), jnp.float32),
        pltpu.SemaphoreType.DMA,
        pltpu.SemaphoreType.DMA,
    ],
    compiler_params=pltpu.CompilerParams(
        use_tc_tiling_on_sc=True, needs_layout_passes=False
    ),
)


_BB = 512


def _mlp_body(e_ref, lv_ref, w1_ref, b1_ref, s1_ref, t1_ref,
              w2_ref, b2_ref, s2_ref, t2_ref,
              w3_ref, b3_ref, s3_ref, t3_ref,
              wout_ref, cout_ref, o_ref):
    h = jnp.dot(e_ref[...], w1_ref[...], preferred_element_type=jnp.float32)
    h = jnp.maximum((h + b1_ref[...]) * s1_ref[...] + t1_ref[...], 0.0)
    h = jnp.dot(h, w2_ref[...], preferred_element_type=jnp.float32)
    h = jnp.maximum((h + b2_ref[...]) * s2_ref[...] + t2_ref[...], 0.0)
    h = jnp.dot(h, w3_ref[...], preferred_element_type=jnp.float32)
    h = jnp.maximum((h + b3_ref[...]) * s3_ref[...] + t3_ref[...], 0.0)
    out = jnp.dot(h, wout_ref[...], preferred_element_type=jnp.float32)
    lr = jnp.sum(lv_ref[...], axis=1, keepdims=True)
    o_ref[...] = out + lr + cout_ref[...]


def kernel(x, lin_table, lin_bias, emb_table, W1, b1, g1, be1,
           W2, b2, g2, be2, W3, b3, g3, be3, Wout, bout):
    offsets = jnp.asarray(_OFFSETS, dtype=x.dtype)
    xo = x + offsets[None, :]
    shape3 = (_NW, _NCH, _CHUNK)
    g = xo.reshape(shape3)
    p = (g // _PACK).astype(jnp.int32)
    sc0 = ((g % _PACK) * _D).astype(jnp.int32)
    flat = jnp.arange(_B * _F, dtype=jnp.int32)
    er = ((flat // _F) % _SPW).reshape(shape3)
    ec0 = ((flat % _F) * _D).reshape(shape3)

    embr = emb_table.reshape(_EMB_ROWS // _PACK, 128)
    lin_flat = lin_table.reshape(-1)

    e, lv = _gather_call(_sc_gather)(embr, lin_flat, p, sc0, er, ec0, g)
    lv = lv.reshape(_B, _F)

    inv = np.float32(1.0) / np.sqrt(np.float32(1.0 + 1e-5))
    s1 = (g1 * inv).reshape(1, -1)
    s2 = (g2 * inv).reshape(1, -1)
    s3 = (g3 * inv).reshape(1, -1)

    h_dim = W1.shape[1]
    full = lambda shape: pl.BlockSpec(shape, lambda i: (0, 0))
    out = pl.pallas_call(
        _mlp_body,
        grid=(_B // _BB,),
        in_specs=[
            pl.BlockSpec((_BB, _F * _D), lambda i: (i, 0)),
            pl.BlockSpec((_BB, _F), lambda i: (i, 0)),
            full((_F * _D, h_dim)), full((1, h_dim)), full((1, h_dim)), full((1, h_dim)),
            full((h_dim, h_dim)), full((1, h_dim)), full((1, h_dim)), full((1, h_dim)),
            full((h_dim, h_dim)), full((1, h_dim)), full((1, h_dim)), full((1, h_dim)),
            full((h_dim, 1)), full((1, 1)),
        ],
        out_specs=pl.BlockSpec((_BB, 1), lambda i: (i, 0)),
        out_shape=jax.ShapeDtypeStruct((_B, 1), jnp.float32),
    )(
        e, lv,
        W1, b1.reshape(1, -1), s1, be1.reshape(1, -1),
        W2, b2.reshape(1, -1), s2, be2.reshape(1, -1),
        W3, b3.reshape(1, -1), s3, be3.reshape(1, -1),
        Wout, (bout + lin_bias).reshape(1, 1),
    )
    return out


# speedup vs baseline: 1.3939x; 1.3939x over previous
import functools

import jax
import jax.numpy as jnp
import numpy as np
from jax import lax
from jax.experimental import pallas as pl
from jax.experimental.pallas import tpu as pltpu
from jax.experimental.pallas import tpu_sc as plsc

_FIELD_DIMS = [100000] * 26
_OFFSETS = np.concatenate(([0], np.cumsum(_FIELD_DIMS)[:-1])).astype(np.int32)

_B = 4096
_F = 26
_D = 16
_NW = 32
_SPW = _B // _NW
_IDX_PER_W = _SPW * _F
_CHUNK = 128
_NCH = _IDX_PER_W // _CHUNK
_EMB_ROWS = 2600000
_PACK = 128 // _D


def _extract(j, buf, p_v, sc0_v, er_v, ec0_v, out_v):
    del p_v
    for grp in range(_CHUNK // 16):
        base = grp * 16
        srow = jnp.arange(16, dtype=jnp.int32) + base
        sc0 = sc0_v[j, pl.ds(base, 16)]
        er = er_v[j, pl.ds(base, 16)]
        ec0 = ec0_v[j, pl.ds(base, 16)]
        for d in range(_D):
            vals = plsc.load_gather(buf, [srow, sc0 + d])
            plsc.store_scatter(out_v, [er, ec0 + d], vals)


def _sc_gather(embr_hbm, lin_hbm, p_hbm, sc0_hbm, er_hbm, ec0_hbm, g_hbm,
               e_out, lv_out,
               p_v, sc0_v, er_v, ec0_v, g_v, buf0, buf1, out_v, lin_v,
               sem_g, sem_l):
    wid = lax.axis_index("s") * 2 + lax.axis_index("c")
    pltpu.sync_copy(p_hbm.at[wid], p_v)
    pltpu.sync_copy(sc0_hbm.at[wid], sc0_v)
    pltpu.sync_copy(er_hbm.at[wid], er_v)
    pltpu.sync_copy(ec0_hbm.at[wid], ec0_v)
    pltpu.sync_copy(g_hbm.at[wid], g_v)

    for j in range(_NCH):
        pltpu.make_async_copy(lin_hbm.at[g_v.at[j]], lin_v.at[j], sem_l).start()

    pltpu.make_async_copy(embr_hbm.at[p_v.at[0]], buf0, sem_g).start()

    def body(k, _):
        j = 2 * k
        pltpu.make_async_copy(embr_hbm.at[p_v.at[j + 1]], buf1, sem_g).start()
        pltpu.make_async_copy(embr_hbm.at[p_v.at[0]], buf0, sem_g).wait()
        _extract(j, buf0, p_v, sc0_v, er_v, ec0_v, out_v)

        @pl.when(k < _NCH // 2 - 1)
        def _():
            pltpu.make_async_copy(embr_hbm.at[p_v.at[j + 2]], buf0, sem_g).start()

        pltpu.make_async_copy(embr_hbm.at[p_v.at[0]], buf1, sem_g).wait()
        _extract(j + 1, buf1, p_v, sc0_v, er_v, ec0_v, out_v)
        return 0

    lax.fori_loop(0, _NCH // 2, body, 0)

    pltpu.make_async_copy(lv_out.at[wid], lin_v, sem_l).wait()
    pltpu.sync_copy(out_v, e_out.at[pl.ds(wid * _SPW, _SPW)])
    pltpu.sync_copy(lin_v, lv_out.at[wid])


_gather_call = functools.partial(
    pl.kernel,
    out_type=[
        jax.ShapeDtypeStruct((_B, _F * _D), jnp.float32),
        jax.ShapeDtypeStruct((_NW, _NCH, _CHUNK), jnp.float32),
    ],
    mesh=plsc.VectorSubcoreMesh(
        core_axis_name="c", subcore_axis_name="s", num_cores=2, num_subcores=16
    ),
    scratch_types=[
        pltpu.VMEM((_NCH, _CHUNK), jnp.int32),
        pltpu.VMEM((_NCH, _CHUNK), jnp.int32),
        pltpu.VMEM((_NCH, _CHUNK), jnp.int32),
        pltpu.VMEM((_NCH, _CHUNK), jnp.int32),
        pltpu.VMEM((_NCH, _CHUNK), jnp.int32),
        pltpu.VMEM((_CHUNK, 128), jnp.float32),
        pltpu.VMEM((_CHUNK, 128), jnp.float32),
        pltpu.VMEM((_SPW, _F * _D), jnp.float32),
        pltpu.VMEM((_NCH, _CHUNK), jnp.float32),
        pltpu.SemaphoreType.DMA,
        pltpu.SemaphoreType.DMA,
    ],
    compiler_params=pltpu.CompilerParams(
        use_tc_tiling_on_sc=True, needs_layout_passes=False
    ),
)


_S = 327680
_RB = 4096
_RG = _S // _RB
_NCB = _S // _RB
_LAST_CB = (_EMB_ROWS - 1) // _RB


def _retile_body(*refs):
    t_refs, o_ref = refs[:_PACK], refs[_PACK]
    for b in range(_PACK):
        o_ref[:, 16 * b:16 * (b + 1)] = jnp.transpose(t_refs[b][...])


def _retile(embT):
    def spec(b):
        return pl.BlockSpec(
            (_D, _RB), lambda i, b=b: (0, jnp.minimum(b * _NCB + i, _LAST_CB))
        )

    return pl.pallas_call(
        _retile_body,
        grid=(_RG,),
        in_specs=[spec(b) for b in range(_PACK)],
        out_specs=pl.BlockSpec((_RB, 128), lambda i: (i, 0)),
        out_shape=jax.ShapeDtypeStruct((_S, 128), jnp.float32),
    )(*([embT] * _PACK))


_BB = 512


def _mlp_body(e_ref, lv_ref, w1_ref, b1_ref, s1_ref, t1_ref,
              w2_ref, b2_ref, s2_ref, t2_ref,
              w3_ref, b3_ref, s3_ref, t3_ref,
              wout_ref, cout_ref, o_ref):
    h = jnp.dot(e_ref[...], w1_ref[...], preferred_element_type=jnp.float32)
    h = jnp.maximum((h + b1_ref[...]) * s1_ref[...] + t1_ref[...], 0.0)
    h = jnp.dot(h, w2_ref[...], preferred_element_type=jnp.float32)
    h = jnp.maximum((h + b2_ref[...]) * s2_ref[...] + t2_ref[...], 0.0)
    h = jnp.dot(h, w3_ref[...], preferred_element_type=jnp.float32)
    h = jnp.maximum((h + b3_ref[...]) * s3_ref[...] + t3_ref[...], 0.0)
    out = jnp.dot(h, wout_ref[...], preferred_element_type=jnp.float32)
    lr = jnp.sum(lv_ref[...], axis=1, keepdims=True)
    o_ref[...] = out + lr + cout_ref[...]


def kernel(x, lin_table, lin_bias, emb_table, W1, b1, g1, be1,
           W2, b2, g2, be2, W3, b3, g3, be3, Wout, bout):
    offsets = jnp.asarray(_OFFSETS, dtype=x.dtype)
    xo = x + offsets[None, :]
    shape3 = (_NW, _NCH, _CHUNK)
    g = xo.reshape(shape3)
    p = (g % _S).astype(jnp.int32)
    sc0 = ((g // _S) * _D).astype(jnp.int32)
    flat = jnp.arange(_B * _F, dtype=jnp.int32)
    er = ((flat // _F) % _SPW).reshape(shape3)
    ec0 = ((flat % _F) * _D).reshape(shape3)

    embr = _retile(emb_table.T)
    lin_flat = lin_table.reshape(-1)

    e, lv = _gather_call(_sc_gather)(embr, lin_flat, p, sc0, er, ec0, g)
    lv = lv.reshape(_B, _F)

    inv = np.float32(1.0) / np.sqrt(np.float32(1.0 + 1e-5))
    s1 = (g1 * inv).reshape(1, -1)
    s2 = (g2 * inv).reshape(1, -1)
    s3 = (g3 * inv).reshape(1, -1)

    h_dim = W1.shape[1]
    full = lambda shape: pl.BlockSpec(shape, lambda i: (0, 0))
    out = pl.pallas_call(
        _mlp_body,
        grid=(_B // _BB,),
        in_specs=[
            pl.BlockSpec((_BB, _F * _D), lambda i: (i, 0)),
            pl.BlockSpec((_BB, _F), lambda i: (i, 0)),
            full((_F * _D, h_dim)), full((1, h_dim)), full((1, h_dim)), full((1, h_dim)),
            full((h_dim, h_dim)), full((1, h_dim)), full((1, h_dim)), full((1, h_dim)),
            full((h_dim, h_dim)), full((1, h_dim)), full((1, h_dim)), full((1, h_dim)),
            full((h_dim, 1)), full((1, 1)),
        ],
        out_specs=pl.BlockSpec((_BB, 1), lambda i: (i, 0)),
        out_shape=jax.ShapeDtypeStruct((_B, 1), jnp.float32),
    )(
        e, lv,
        W1, b1.reshape(1, -1), s1, be1.reshape(1, -1),
        W2, b2.reshape(1, -1), s2, be2.reshape(1, -1),
        W3, b3.reshape(1, -1), s3, be3.reshape(1, -1),
        Wout, (bout + lin_bias).reshape(1, 1),
    )
    return out


# speedup vs baseline: 3.6095x; 2.5895x over previous
import functools

import jax
import jax.numpy as jnp
import numpy as np
from jax import lax
from jax.experimental import pallas as pl
from jax.experimental.pallas import tpu as pltpu
from jax.experimental.pallas import tpu_sc as plsc

_FIELD_DIMS = [100000] * 26
_OFFSETS = np.concatenate(([0], np.cumsum(_FIELD_DIMS)[:-1])).astype(np.int32)

_B = 4096
_F = 26
_D = 16
_NW = 32
_SPW = _B // _NW
_IDX_PER_W = _SPW * _F
_CHUNK = 128
_NCH = _IDX_PER_W // _CHUNK
_EMB_ROWS = 2600000
_PACK = 128 // _D


def _extract(j, buf, p_v, sc0_v, er_v, ec0_v, out_v):
    del p_v
    for grp in range(_CHUNK // 16):
        base = grp * 16
        srow = jnp.arange(16, dtype=jnp.int32) + base
        sc0 = sc0_v[j, pl.ds(base, 16)]
        er = er_v[j, pl.ds(base, 16)]
        ec0 = ec0_v[j, pl.ds(base, 16)]
        for d in range(_D):
            vals = plsc.load_gather(buf, [srow, sc0 + d])
            plsc.store_scatter(out_v, [er, ec0 + d], vals)


def _sc_gather(embr_hbm, lin_hbm, p_hbm, sc0_hbm, er_hbm, ec0_hbm, g_hbm,
               e_out, lv_out,
               p_v, sc0_v, er_v, ec0_v, g_v, buf0, buf1, out_v, lin_v,
               sem_g, sem_l):
    wid = lax.axis_index("s") * 2 + lax.axis_index("c")
    pltpu.sync_copy(p_hbm.at[wid], p_v)
    pltpu.sync_copy(sc0_hbm.at[wid], sc0_v)
    pltpu.sync_copy(er_hbm.at[wid], er_v)
    pltpu.sync_copy(ec0_hbm.at[wid], ec0_v)
    pltpu.sync_copy(g_hbm.at[wid], g_v)

    for j in range(_NCH):
        pltpu.make_async_copy(lin_hbm.at[g_v.at[j]], lin_v.at[j], sem_l).start()

    pltpu.make_async_copy(embr_hbm.at[p_v.at[0]], buf0, sem_g).start()

    def body(k, _):
        j = 2 * k
        pltpu.make_async_copy(embr_hbm.at[p_v.at[j + 1]], buf1, sem_g).start()
        pltpu.make_async_copy(embr_hbm.at[p_v.at[0]], buf0, sem_g).wait()
        _extract(j, buf0, p_v, sc0_v, er_v, ec0_v, out_v)

        @pl.when(k < _NCH // 2 - 1)
        def _():
            pltpu.make_async_copy(embr_hbm.at[p_v.at[j + 2]], buf0, sem_g).start()

        pltpu.make_async_copy(embr_hbm.at[p_v.at[0]], buf1, sem_g).wait()
        _extract(j + 1, buf1, p_v, sc0_v, er_v, ec0_v, out_v)
        return 0

    lax.fori_loop(0, _NCH // 2, body, 0)

    pltpu.make_async_copy(lv_out.at[wid], lin_v, sem_l).wait()
    pltpu.sync_copy(out_v, e_out.at[pl.ds(wid * _SPW, _SPW)])
    pltpu.sync_copy(lin_v, lv_out.at[wid])


_gather_call = functools.partial(
    pl.kernel,
    out_type=[
        jax.ShapeDtypeStruct((_B, _F * _D), jnp.float32),
        jax.ShapeDtypeStruct((_NW, _NCH, _CHUNK), jnp.float32),
    ],
    mesh=plsc.VectorSubcoreMesh(
        core_axis_name="c", subcore_axis_name="s", num_cores=2, num_subcores=16
    ),
    scratch_types=[
        pltpu.VMEM((_NCH, _CHUNK), jnp.int32),
        pltpu.VMEM((_NCH, _CHUNK), jnp.int32),
        pltpu.VMEM((_NCH, _CHUNK), jnp.int32),
        pltpu.VMEM((_NCH, _CHUNK), jnp.int32),
        pltpu.VMEM((_NCH, _CHUNK), jnp.int32),
        pltpu.VMEM((_CHUNK, 128), jnp.float32),
        pltpu.VMEM((_CHUNK, 128), jnp.float32),
        pltpu.VMEM((_SPW, _F * _D), jnp.float32),
        pltpu.VMEM((_NCH, _CHUNK), jnp.float32),
        pltpu.SemaphoreType.DMA,
        pltpu.SemaphoreType.DMA,
    ],
    compiler_params=pltpu.CompilerParams(
        use_tc_tiling_on_sc=True, needs_layout_passes=False
    ),
)


_S = 327680
_RB = 4096
_RG = _S // _RB
_NCB = _S // _RB
_LAST_CB = (_EMB_ROWS - 1) // _RB


def _retile_body(*refs):
    t_refs, o_ref = refs[:_PACK], refs[_PACK]
    eye128 = jnp.eye(128, dtype=jnp.float32)
    t_cat = jnp.concatenate([t_refs[b][...] for b in range(_PACK)], axis=0)
    o_ref[...] = jax.lax.dot_general(
        t_cat, eye128, (((0,), (0,)), ((), ())),
        preferred_element_type=jnp.float32,
    )


def _retile(embT):
    def spec(b):
        return pl.BlockSpec(
            (_D, _RB), lambda i, b=b: (0, jnp.minimum(b * _NCB + i, _LAST_CB))
        )

    return pl.pallas_call(
        _retile_body,
        grid=(_RG,),
        in_specs=[spec(b) for b in range(_PACK)],
        out_specs=pl.BlockSpec((_RB, 128), lambda i: (i, 0)),
        out_shape=jax.ShapeDtypeStruct((_S, 128), jnp.float32),
        compiler_params=pltpu.CompilerParams(fuse_transposed_lhs_in_matmul=True),
    )(*([embT] * _PACK))


_BB = 512


def _mlp_body(e_ref, lv_ref, w1_ref, b1_ref, s1_ref, t1_ref,
              w2_ref, b2_ref, s2_ref, t2_ref,
              w3_ref, b3_ref, s3_ref, t3_ref,
              wout_ref, cout_ref, o_ref):
    h = jnp.dot(e_ref[...], w1_ref[...], preferred_element_type=jnp.float32)
    h = jnp.maximum((h + b1_ref[...]) * s1_ref[...] + t1_ref[...], 0.0)
    h = jnp.dot(h, w2_ref[...], preferred_element_type=jnp.float32)
    h = jnp.maximum((h + b2_ref[...]) * s2_ref[...] + t2_ref[...], 0.0)
    h = jnp.dot(h, w3_ref[...], preferred_element_type=jnp.float32)
    h = jnp.maximum((h + b3_ref[...]) * s3_ref[...] + t3_ref[...], 0.0)
    out = jnp.dot(h, wout_ref[...], preferred_element_type=jnp.float32)
    lr = jnp.sum(lv_ref[...], axis=1, keepdims=True)
    o_ref[...] = out + lr + cout_ref[...]


def kernel(x, lin_table, lin_bias, emb_table, W1, b1, g1, be1,
           W2, b2, g2, be2, W3, b3, g3, be3, Wout, bout):
    offsets = jnp.asarray(_OFFSETS, dtype=x.dtype)
    xo = x + offsets[None, :]
    shape3 = (_NW, _NCH, _CHUNK)
    g = xo.reshape(shape3)
    p = (g % _S).astype(jnp.int32)
    sc0 = ((g // _S) * _D).astype(jnp.int32)
    flat = jnp.arange(_B * _F, dtype=jnp.int32)
    er = ((flat // _F) % _SPW).reshape(shape3)
    ec0 = ((flat % _F) * _D).reshape(shape3)

    embr = _retile(emb_table.T)
    lin_flat = lin_table.reshape(-1)

    e, lv = _gather_call(_sc_gather)(embr, lin_flat, p, sc0, er, ec0, g)
    lv = lv.reshape(_B, _F)

    inv = np.float32(1.0) / np.sqrt(np.float32(1.0 + 1e-5))
    s1 = (g1 * inv).reshape(1, -1)
    s2 = (g2 * inv).reshape(1, -1)
    s3 = (g3 * inv).reshape(1, -1)

    h_dim = W1.shape[1]
    full = lambda shape: pl.BlockSpec(shape, lambda i: (0, 0))
    out = pl.pallas_call(
        _mlp_body,
        grid=(_B // _BB,),
        in_specs=[
            pl.BlockSpec((_BB, _F * _D), lambda i: (i, 0)),
            pl.BlockSpec((_BB, _F), lambda i: (i, 0)),
            full((_F * _D, h_dim)), full((1, h_dim)), full((1, h_dim)), full((1, h_dim)),
            full((h_dim, h_dim)), full((1, h_dim)), full((1, h_dim)), full((1, h_dim)),
            full((h_dim, h_dim)), full((1, h_dim)), full((1, h_dim)), full((1, h_dim)),
            full((h_dim, 1)), full((1, 1)),
        ],
        out_specs=pl.BlockSpec((_BB, 1), lambda i: (i, 0)),
        out_shape=jax.ShapeDtypeStruct((_B, 1), jnp.float32),
    )(
        e, lv,
        W1, b1.reshape(1, -1), s1, be1.reshape(1, -1),
        W2, b2.reshape(1, -1), s2, be2.reshape(1, -1),
        W3, b3.reshape(1, -1), s3, be3.reshape(1, -1),
        Wout, (bout + lin_bias).reshape(1, 1),
    )
    return out


# speedup vs baseline: 4.3637x; 1.2089x over previous
import functools

import jax
import jax.numpy as jnp
import numpy as np
from jax import lax
from jax.experimental import pallas as pl
from jax.experimental.pallas import tpu as pltpu
from jax.experimental.pallas import tpu_sc as plsc

_FIELD_DIMS = [100000] * 26
_OFFSETS = np.concatenate(([0], np.cumsum(_FIELD_DIMS)[:-1])).astype(np.int32)

_B = 4096
_F = 26
_D = 16
_NW = 32
_SPW = _B // _NW
_IDX_PER_W = _SPW * _F
_CHUNK = 128
_NCH = _IDX_PER_W // _CHUNK
_EMB_ROWS = 2600000
_PACK = 128 // _D


def _extract(j, buf, p_v, sc0_v, er_v, ec0_v, out_v):
    del p_v
    for grp in range(_CHUNK // 16):
        base = grp * 16
        srow = jnp.arange(16, dtype=jnp.int32) + base
        sc0 = sc0_v[j, pl.ds(base, 16)]
        er = er_v[j, pl.ds(base, 16)]
        ec0 = ec0_v[j, pl.ds(base, 16)]
        for d in range(_D):
            vals = plsc.load_gather(buf, [srow, sc0 + d])
            plsc.store_scatter(out_v, [er, ec0 + d], vals)


def _sc_gather(embr_hbm, lin_hbm, p_hbm, sc0_hbm, er_hbm, ec0_hbm, g_hbm,
               e_out, lv_out,
               p_v, sc0_v, er_v, ec0_v, g_v, buf0, buf1, out_v, lin_v,
               sem_g, sem_l):
    wid = lax.axis_index("s") * 2 + lax.axis_index("c")
    pltpu.sync_copy(p_hbm.at[wid], p_v)
    pltpu.sync_copy(sc0_hbm.at[wid], sc0_v)
    pltpu.sync_copy(er_hbm.at[wid], er_v)
    pltpu.sync_copy(ec0_hbm.at[wid], ec0_v)
    pltpu.sync_copy(g_hbm.at[wid], g_v)

    for j in range(_NCH):
        pltpu.make_async_copy(lin_hbm.at[g_v.at[j]], lin_v.at[j], sem_l).start()

    pltpu.make_async_copy(embr_hbm.at[p_v.at[0]], buf0, sem_g).start()

    def body(k, _):
        j = 2 * k
        pltpu.make_async_copy(embr_hbm.at[p_v.at[j + 1]], buf1, sem_g).start()
        pltpu.make_async_copy(embr_hbm.at[p_v.at[0]], buf0, sem_g).wait()
        _extract(j, buf0, p_v, sc0_v, er_v, ec0_v, out_v)

        @pl.when(k < _NCH // 2 - 1)
        def _():
            pltpu.make_async_copy(embr_hbm.at[p_v.at[j + 2]], buf0, sem_g).start()

        pltpu.make_async_copy(embr_hbm.at[p_v.at[0]], buf1, sem_g).wait()
        _extract(j + 1, buf1, p_v, sc0_v, er_v, ec0_v, out_v)
        return 0

    lax.fori_loop(0, _NCH // 2, body, 0)

    pltpu.make_async_copy(lv_out.at[wid], lin_v, sem_l).wait()
    pltpu.sync_copy(out_v, e_out.at[pl.ds(wid * _SPW, _SPW)])
    pltpu.sync_copy(lin_v, lv_out.at[wid])


_gather_call = functools.partial(
    pl.kernel,
    out_type=[
        jax.ShapeDtypeStruct((_B, _F * _D), jnp.float32),
        jax.ShapeDtypeStruct((_NW, _NCH, _CHUNK), jnp.float32),
    ],
    mesh=plsc.VectorSubcoreMesh(
        core_axis_name="c", subcore_axis_name="s", num_cores=2, num_subcores=16
    ),
    scratch_types=[
        pltpu.VMEM((_NCH, _CHUNK), jnp.int32),
        pltpu.VMEM((_NCH, _CHUNK), jnp.int32),
        pltpu.VMEM((_NCH, _CHUNK), jnp.int32),
        pltpu.VMEM((_NCH, _CHUNK), jnp.int32),
        pltpu.VMEM((_NCH, _CHUNK), jnp.int32),
        pltpu.VMEM((_CHUNK, 128), jnp.float32),
        pltpu.VMEM((_CHUNK, 128), jnp.float32),
        pltpu.VMEM((_SPW, _F * _D), jnp.float32),
        pltpu.VMEM((_NCH, _CHUNK), jnp.float32),
        pltpu.SemaphoreType.DMA,
        pltpu.SemaphoreType.DMA,
    ],
    compiler_params=pltpu.CompilerParams(
        use_tc_tiling_on_sc=True, needs_layout_passes=False
    ),
)


_S = 327680
_RB = 4096
_RG = _S // _RB
_NCB = _S // _RB
_LAST_CB = (_EMB_ROWS - 1) // _RB


def _retile_body(*refs):
    t_refs, o_ref = refs[:_PACK], refs[_PACK]
    eye128 = jnp.eye(128, dtype=jnp.float32)
    t_cat = jnp.concatenate([t_refs[b][...] for b in range(_PACK)], axis=0)
    o_ref[...] = jax.lax.dot_general(
        t_cat, eye128, (((0,), (0,)), ((), ())),
        preferred_element_type=jnp.float32,
    )


def _retile(embT):
    def spec(b):
        return pl.BlockSpec(
            (_D, _RB), lambda i, b=b: (0, jnp.minimum(b * _NCB + i, _LAST_CB))
        )

    return pl.pallas_call(
        _retile_body,
        grid=(_RG,),
        in_specs=[spec(b) for b in range(_PACK)],
        out_specs=pl.BlockSpec((_RB, 128), lambda i: (i, 0)),
        out_shape=jax.ShapeDtypeStruct((_S, 128), jnp.float32),
        compiler_params=pltpu.CompilerParams(fuse_transposed_lhs_in_matmul=True),
    )(*([embT] * _PACK))


_LB = 32768


def _lin_flatten_body(l_ref, o_ref):
    o_ref[...] = l_ref[0, :]


def _lin_flatten(linT):
    n = linT.shape[1]
    ng = (n + _LB - 1) // _LB
    out = pl.pallas_call(
        _lin_flatten_body,
        grid=(ng,),
        in_specs=[pl.BlockSpec((1, _LB), lambda i: (0, i))],
        out_specs=pl.BlockSpec((_LB,), lambda i: (i,)),
        out_shape=jax.ShapeDtypeStruct((ng * _LB,), jnp.float32),
    )(linT)
    return out[:n]


_BB = 512


def _mlp_body(e_ref, lv_ref, w1_ref, b1_ref, s1_ref, t1_ref,
              w2_ref, b2_ref, s2_ref, t2_ref,
              w3_ref, b3_ref, s3_ref, t3_ref,
              wout_ref, cout_ref, o_ref):
    h = jnp.dot(e_ref[...], w1_ref[...], preferred_element_type=jnp.float32)
    h = jnp.maximum((h + b1_ref[...]) * s1_ref[...] + t1_ref[...], 0.0)
    h = jnp.dot(h, w2_ref[...], preferred_element_type=jnp.float32)
    h = jnp.maximum((h + b2_ref[...]) * s2_ref[...] + t2_ref[...], 0.0)
    h = jnp.dot(h, w3_ref[...], preferred_element_type=jnp.float32)
    h = jnp.maximum((h + b3_ref[...]) * s3_ref[...] + t3_ref[...], 0.0)
    out = jnp.dot(h, wout_ref[...], preferred_element_type=jnp.float32)
    lr = jnp.sum(lv_ref[...], axis=1, keepdims=True)
    o_ref[...] = out + lr + cout_ref[...]


def kernel(x, lin_table, lin_bias, emb_table, W1, b1, g1, be1,
           W2, b2, g2, be2, W3, b3, g3, be3, Wout, bout):
    offsets = jnp.asarray(_OFFSETS, dtype=x.dtype)
    xo = x + offsets[None, :]
    shape3 = (_NW, _NCH, _CHUNK)
    g = xo.reshape(shape3)
    p = (g % _S).astype(jnp.int32)
    sc0 = ((g // _S) * _D).astype(jnp.int32)
    flat = jnp.arange(_B * _F, dtype=jnp.int32)
    er = ((flat // _F) % _SPW).reshape(shape3)
    ec0 = ((flat % _F) * _D).reshape(shape3)

    embr = _retile(emb_table.T)
    lin_flat = _lin_flatten(lin_table.T)

    e, lv = _gather_call(_sc_gather)(embr, lin_flat, p, sc0, er, ec0, g)
    lv = lv.reshape(_B, _F)

    inv = np.float32(1.0) / np.sqrt(np.float32(1.0 + 1e-5))
    s1 = (g1 * inv).reshape(1, -1)
    s2 = (g2 * inv).reshape(1, -1)
    s3 = (g3 * inv).reshape(1, -1)

    h_dim = W1.shape[1]
    full = lambda shape: pl.BlockSpec(shape, lambda i: (0, 0))
    out = pl.pallas_call(
        _mlp_body,
        grid=(_B // _BB,),
        in_specs=[
            pl.BlockSpec((_BB, _F * _D), lambda i: (i, 0)),
            pl.BlockSpec((_BB, _F), lambda i: (i, 0)),
            full((_F * _D, h_dim)), full((1, h_dim)), full((1, h_dim)), full((1, h_dim)),
            full((h_dim, h_dim)), full((1, h_dim)), full((1, h_dim)), full((1, h_dim)),
            full((h_dim, h_dim)), full((1, h_dim)), full((1, h_dim)), full((1, h_dim)),
            full((h_dim, 1)), full((1, 1)),
        ],
        out_specs=pl.BlockSpec((_BB, 1), lambda i: (i, 0)),
        out_shape=jax.ShapeDtypeStruct((_B, 1), jnp.float32),
    )(
        e, lv,
        W1, b1.reshape(1, -1), s1, be1.reshape(1, -1),
        W2, b2.reshape(1, -1), s2, be2.reshape(1, -1),
        W3, b3.reshape(1, -1), s3, be3.reshape(1, -1),
        Wout, (bout + lin_bias).reshape(1, 1),
    )
    return out


# speedup vs baseline: 5.2645x; 1.2064x over previous
import functools

import jax
import jax.numpy as jnp
import numpy as np
from jax import lax
from jax.experimental import pallas as pl
from jax.experimental.pallas import tpu as pltpu
from jax.experimental.pallas import tpu_sc as plsc

_FIELD_DIMS = [100000] * 26
_OFFSETS = np.concatenate(([0], np.cumsum(_FIELD_DIMS)[:-1])).astype(np.int32)

_B = 4096
_F = 26
_D = 16
_NW = 32
_SPW = _B // _NW
_IDX_PER_W = _SPW * _F
_CHUNK = 128
_NCH = _IDX_PER_W // _CHUNK
_EMB_ROWS = 2600000
_PACK = 128 // _D


def _extract(j, buf, p_v, sc0_v, er_v, ec0_v, out_v):
    del p_v
    for grp in range(_CHUNK // 16):
        base = grp * 16
        srow = jnp.arange(16, dtype=jnp.int32) + base
        sc0 = sc0_v[j, pl.ds(base, 16)]
        er = er_v[j, pl.ds(base, 16)]
        ec0 = ec0_v[j, pl.ds(base, 16)]
        for d in range(_D):
            vals = plsc.load_gather(buf, [srow, sc0 + d])
            plsc.store_scatter(out_v, [er, ec0 + d], vals)


def _sc_gather(embr_hbm, lin_hbm, p_hbm, sc0_hbm, er_hbm, ec0_hbm, g_hbm,
               e_out, lv_out,
               p_v, sc0_v, er_v, ec0_v, g_v, buf0, buf1, out_v, lin_v,
               sem_g, sem_l):
    wid = lax.axis_index("s") * 2 + lax.axis_index("c")
    pltpu.sync_copy(p_hbm.at[wid], p_v)
    pltpu.sync_copy(sc0_hbm.at[wid], sc0_v)
    pltpu.sync_copy(er_hbm.at[wid], er_v)
    pltpu.sync_copy(ec0_hbm.at[wid], ec0_v)
    pltpu.sync_copy(g_hbm.at[wid], g_v)

    for j in range(_NCH):
        pltpu.make_async_copy(lin_hbm.at[g_v.at[j]], lin_v.at[j], sem_l).start()

    pltpu.make_async_copy(embr_hbm.at[p_v.at[0]], buf0, sem_g).start()

    def body(k, _):
        j = 2 * k
        pltpu.make_async_copy(embr_hbm.at[p_v.at[j + 1]], buf1, sem_g).start()
        pltpu.make_async_copy(embr_hbm.at[p_v.at[0]], buf0, sem_g).wait()
        _extract(j, buf0, p_v, sc0_v, er_v, ec0_v, out_v)

        @pl.when(k < _NCH // 2 - 1)
        def _():
            pltpu.make_async_copy(embr_hbm.at[p_v.at[j + 2]], buf0, sem_g).start()

        pltpu.make_async_copy(embr_hbm.at[p_v.at[0]], buf1, sem_g).wait()
        _extract(j + 1, buf1, p_v, sc0_v, er_v, ec0_v, out_v)
        return 0

    lax.fori_loop(0, _NCH // 2, body, 0)

    pltpu.make_async_copy(lv_out.at[wid], lin_v, sem_l).wait()
    pltpu.sync_copy(out_v, e_out.at[pl.ds(wid * _SPW, _SPW)])
    pltpu.sync_copy(lin_v, lv_out.at[wid])


_gather_call = functools.partial(
    pl.kernel,
    out_type=[
        jax.ShapeDtypeStruct((_B, _F * _D), jnp.float32),
        jax.ShapeDtypeStruct((_NW, _NCH, _CHUNK), jnp.float32),
    ],
    mesh=plsc.VectorSubcoreMesh(
        core_axis_name="c", subcore_axis_name="s", num_cores=2, num_subcores=16
    ),
    scratch_types=[
        pltpu.VMEM((_NCH, _CHUNK), jnp.int32),
        pltpu.VMEM((_NCH, _CHUNK), jnp.int32),
        pltpu.VMEM((_NCH, _CHUNK), jnp.int32),
        pltpu.VMEM((_NCH, _CHUNK), jnp.int32),
        pltpu.VMEM((_NCH, _CHUNK), jnp.int32),
        pltpu.VMEM((_CHUNK, 128), jnp.float32),
        pltpu.VMEM((_CHUNK, 128), jnp.float32),
        pltpu.VMEM((_SPW, _F * _D), jnp.float32),
        pltpu.VMEM((_NCH, _CHUNK), jnp.float32),
        pltpu.SemaphoreType.DMA,
        pltpu.SemaphoreType.DMA,
    ],
    compiler_params=pltpu.CompilerParams(
        use_tc_tiling_on_sc=True, needs_layout_passes=False
    ),
)


_S = 327680
_RB = 4096
_RG = _S // _RB
_NCB = _S // _RB
_LAST_CB = (_EMB_ROWS - 1) // _RB


_LB = _PACK * _RB


def _retile_body(*refs):
    t_refs, lin_ref, o_ref, lo_ref = refs[:_PACK], refs[_PACK], refs[_PACK + 1], refs[_PACK + 2]
    eye128 = jnp.eye(128, dtype=jnp.float32)
    t_cat = jnp.concatenate([t_refs[b][...] for b in range(_PACK)], axis=0)
    o_ref[...] = jax.lax.dot_general(
        t_cat, eye128, (((0,), (0,)), ((), ())),
        preferred_element_type=jnp.float32,
    )
    lo_ref[...] = lin_ref[0, :]


def _retile(embT, linT):
    def spec(b):
        return pl.BlockSpec(
            (_D, _RB), lambda i, b=b: (0, jnp.minimum(b * _NCB + i, _LAST_CB))
        )

    return pl.pallas_call(
        _retile_body,
        grid=(_RG,),
        in_specs=[spec(b) for b in range(_PACK)]
        + [pl.BlockSpec((1, _LB), lambda i: (0, i))],
        out_specs=[
            pl.BlockSpec((_RB, 128), lambda i: (i, 0)),
            pl.BlockSpec((_LB,), lambda i: (i,)),
        ],
        out_shape=[
            jax.ShapeDtypeStruct((_S, 128), jnp.float32),
            jax.ShapeDtypeStruct((_EMB_ROWS,), jnp.float32),
        ],
        compiler_params=pltpu.CompilerParams(fuse_transposed_lhs_in_matmul=True),
    )(*([embT] * _PACK + [linT]))


_BB = 512


def _mlp_body(e_ref, lv_ref, w1_ref, b1_ref, s1_ref, t1_ref,
              w2_ref, b2_ref, s2_ref, t2_ref,
              w3_ref, b3_ref, s3_ref, t3_ref,
              wout_ref, cout_ref, o_ref):
    h = jnp.dot(e_ref[...], w1_ref[...], preferred_element_type=jnp.float32)
    h = jnp.maximum((h + b1_ref[...]) * s1_ref[...] + t1_ref[...], 0.0)
    h = jnp.dot(h, w2_ref[...], preferred_element_type=jnp.float32)
    h = jnp.maximum((h + b2_ref[...]) * s2_ref[...] + t2_ref[...], 0.0)
    h = jnp.dot(h, w3_ref[...], preferred_element_type=jnp.float32)
    h = jnp.maximum((h + b3_ref[...]) * s3_ref[...] + t3_ref[...], 0.0)
    out = jnp.dot(h, wout_ref[...], preferred_element_type=jnp.float32)
    lr = jnp.sum(lv_ref[...], axis=1, keepdims=True)
    o_ref[...] = out + lr + cout_ref[...]


def kernel(x, lin_table, lin_bias, emb_table, W1, b1, g1, be1,
           W2, b2, g2, be2, W3, b3, g3, be3, Wout, bout):
    offsets = jnp.asarray(_OFFSETS, dtype=x.dtype)
    xo = x + offsets[None, :]
    shape3 = (_NW, _NCH, _CHUNK)
    g = xo.reshape(shape3)
    p = (g % _S).astype(jnp.int32)
    sc0 = ((g // _S) * _D).astype(jnp.int32)
    flat = jnp.arange(_B * _F, dtype=jnp.int32)
    er = ((flat // _F) % _SPW).reshape(shape3)
    ec0 = ((flat % _F) * _D).reshape(shape3)

    embr, lin_flat = _retile(emb_table.T, lin_table.T)

    e, lv = _gather_call(_sc_gather)(embr, lin_flat, p, sc0, er, ec0, g)
    lv = lv.reshape(_B, _F)

    inv = np.float32(1.0) / np.sqrt(np.float32(1.0 + 1e-5))
    s1 = (g1 * inv).reshape(1, -1)
    s2 = (g2 * inv).reshape(1, -1)
    s3 = (g3 * inv).reshape(1, -1)

    h_dim = W1.shape[1]
    full = lambda shape: pl.BlockSpec(shape, lambda i: (0, 0))
    out = pl.pallas_call(
        _mlp_body,
        grid=(_B // _BB,),
        in_specs=[
            pl.BlockSpec((_BB, _F * _D), lambda i: (i, 0)),
            pl.BlockSpec((_BB, _F), lambda i: (i, 0)),
            full((_F * _D, h_dim)), full((1, h_dim)), full((1, h_dim)), full((1, h_dim)),
            full((h_dim, h_dim)), full((1, h_dim)), full((1, h_dim)), full((1, h_dim)),
            full((h_dim, h_dim)), full((1, h_dim)), full((1, h_dim)), full((1, h_dim)),
            full((h_dim, 1)), full((1, 1)),
        ],
        out_specs=pl.BlockSpec((_BB, 1), lambda i: (i, 0)),
        out_shape=jax.ShapeDtypeStruct((_B, 1), jnp.float32),
    )(
        e, lv,
        W1, b1.reshape(1, -1), s1, be1.reshape(1, -1),
        W2, b2.reshape(1, -1), s2, be2.reshape(1, -1),
        W3, b3.reshape(1, -1), s3, be3.reshape(1, -1),
        Wout, (bout + lin_bias).reshape(1, 1),
    )
    return out
